# baseline (speedup 1.0000x reference)
import functools

import jax
import jax.numpy as jnp
from jax.experimental import pallas as pl
from jax.experimental.pallas import tpu as pltpu

K_PER = 256
Q_MIN = 0.5
S_B = 1.0
HUBER_SCALE = 2.0
E_DEN_OFF = 1.0
PAYLOAD_REL_THR = 0.1


def _atanh(x):
    return 0.5 * jnp.log((1.0 + x) / (1.0 - x))


def _oc_kernel(rs_ref, beta_ref, cc_ref, pe_ref, pp_ref, pt_ref, pid_ref,
               tidx_ref, te_ref, tt_ref,
               out_ref,
               mb_ref, mx0_ref, mx1_ref, qa_ref, thr_ref,
               acc_ref, pws_ref, pww_ref, ssum_ref,
               *, B, K, NB, N):
    p = pl.program_id(0)
    b = pl.program_id(1)

    kio = jax.lax.broadcasted_iota(jnp.int32, (1, K), 1)
    hid = jax.lax.broadcasted_iota(jnp.int32, (B, 1), 0) + b * B
    ev = (hid >= rs_ref[1]).astype(jnp.int32)
    tid = tidx_ref[:, 0:1]
    g = jnp.where(tid >= 0, tid + ev * K_PER, -1)
    beta = jnp.clip(beta_ref[:, 0:1], 1e-6, 1.0 - 1e-4)
    M = g == kio
    x0 = cc_ref[:, 0:1]
    x1 = cc_ref[:, 1:2]

    @pl.when(p == 0)
    def _phase0():
        @pl.when(b == 0)
        def _init():
            mb_ref[...] = jnp.full((1, K), -1.0, jnp.float32)
            mx0_ref[...] = jnp.zeros((1, K), jnp.float32)
            mx1_ref[...] = jnp.zeros((1, K), jnp.float32)
            acc_ref[...] = jnp.zeros((1, K), jnp.float32)
            pws_ref[...] = jnp.zeros((1, K), jnp.float32)
            pww_ref[...] = jnp.zeros((1, K), jnp.float32)
            ssum_ref[0] = 0.0
            ssum_ref[1] = 0.0

        mbk = jnp.where(M, beta, -1.0)
        bmax = jnp.max(mbk, axis=0, keepdims=True)
        ii = jax.lax.broadcasted_iota(jnp.int32, (B, K), 0)
        win = M & (mbk == bmax)
        iwin = jnp.min(jnp.where(win, ii, N), axis=0, keepdims=True)
        uniq = win & (ii == iwin)
        bx0 = jnp.sum(jnp.where(uniq, x0, 0.0), axis=0, keepdims=True)
        bx1 = jnp.sum(jnp.where(uniq, x1, 0.0), axis=0, keepdims=True)
        upd = bmax > mb_ref[...]
        mb_ref[...] = jnp.where(upd, bmax, mb_ref[...])
        mx0_ref[...] = jnp.where(upd, bx0, mx0_ref[...])
        mx1_ref[...] = jnp.where(upd, bx1, mx1_ref[...])

        nmask = tid < 0
        ssum_ref[0] += jnp.sum(jnp.where(nmask, beta, 0.0))
        ssum_ref[1] += jnp.sum(nmask.astype(jnp.float32))

    @pl.when(p == 1)
    def _phase1():
        @pl.when(b == 0)
        def _prep():
            mb = mb_ref[...]
            exists = mb > 0.0
            ba = jnp.where(exists, mb, 0.0)
            at = _atanh(ba)
            qa_ref[...] = jnp.where(exists, at * at + Q_MIN, 0.0)
            thr_ref[...] = PAYLOAD_REL_THR * ba

        dx0 = x0 - mx0_ref[...]
        dx1 = x1 - mx1_ref[...]
        d2 = dx0 * dx0 + dx1 * dx1
        d = jnp.sqrt(d2 + 1e-9)
        rw = jnp.maximum(1.0 - d, 0.0)
        same_ev = ev == (kio // K_PER)
        sel = jnp.where(M, d2, jnp.where(same_ev, rw, 0.0))
        at = _atanh(beta)
        q = at * at + Q_MIN
        acc_ref[...] += jnp.sum((q * qa_ref[...]) * sel, axis=0, keepdims=True)

        te = te_ref[:, 0:1]
        ew = jnp.where(te > 10.0, 1.0, (te / 10.0 + 0.1) / 1.1)
        ste = jnp.sqrt(te + 0.001)
        l = jnp.abs(te - pe_ref[:, 0:1]) / (ste + E_DEN_OFF)
        delta = ste * HUBER_SCALE
        el = jnp.where(l <= delta, 0.5 * l * l, delta * (l - 0.5 * delta))
        dp0 = tt_ref[:, 1:2] - pp_ref[:, 0:1]
        dp1 = tt_ref[:, 2:3] - pp_ref[:, 1:2]
        posl = (dp0 * dp0 + dp1 * dp1) / 100.0
        dtim = tt_ref[:, 0:1] * 1e9 - pt_ref[:, 0:1]
        timl = dtim * dtim
        cls = (1e-8 / 6.0) * jnp.sum(pid_ref[...] * pid_ref[...], axis=1,
                                     keepdims=True)
        wsum = (el + posl + timl + cls) * ew

        pwv = jnp.where(M & (beta > thr_ref[...]), beta, 0.0)
        pws_ref[...] += jnp.sum(pwv, axis=0, keepdims=True)
        pww_ref[...] += jnp.sum(pwv * wsum, axis=0, keepdims=True)

        @pl.when(b == NB - 1)
        def _fin():
            mb = mb_ref[...]
            exists = (mb > 0.0).astype(jnp.float32)
            n_obj = jnp.maximum(jnp.sum(exists), 1.0)
            l_minb = jnp.sum(exists * (1.0 - mb)) / n_obj
            l_pay = jnp.sum(exists * pww_ref[...]
                            / (pws_ref[...] + 1e-9)) / n_obj
            pair = jnp.sum(acc_ref[...]) / N
            l_noise = S_B * ssum_ref[0] / jnp.maximum(ssum_ref[1], 1.0)
            out_ref[...] = (pair + l_minb + l_noise + l_pay).reshape(1, 1)


def kernel(pred_beta, pred_ccoords, pred_energy, pred_pos, pred_time,
           pred_id, t_idx, t_energy, t_pos, t_time, t_pid, rowsplits):
    n = pred_beta.shape[0]
    n_events = rowsplits.shape[0] - 1
    k_tot = n_events * K_PER
    B = 1024
    NB = n // B
    tt = jnp.concatenate([t_time, t_pos], axis=1)

    hspec = lambda c: pl.BlockSpec((B, c), lambda p, b: (b, 0))
    out = pl.pallas_call(
        functools.partial(_oc_kernel, B=B, K=k_tot, NB=NB, N=n),
        grid=(2, NB),
        in_specs=[
            pl.BlockSpec(memory_space=pltpu.SMEM),
            hspec(1), hspec(2), hspec(1), hspec(2), hspec(1), hspec(6),
            hspec(1), hspec(1), hspec(3),
        ],
        out_specs=pl.BlockSpec((1, 1), lambda p, b: (0, 0)),
        out_shape=jax.ShapeDtypeStruct((1, 1), jnp.float32),
        scratch_shapes=[
            pltpu.VMEM((1, k_tot), jnp.float32),
            pltpu.VMEM((1, k_tot), jnp.float32),
            pltpu.VMEM((1, k_tot), jnp.float32),
            pltpu.VMEM((1, k_tot), jnp.float32),
            pltpu.VMEM((1, k_tot), jnp.float32),
            pltpu.VMEM((1, k_tot), jnp.float32),
            pltpu.VMEM((1, k_tot), jnp.float32),
            pltpu.VMEM((1, k_tot), jnp.float32),
            pltpu.SMEM((2,), jnp.float32),
        ],
        compiler_params=pltpu.CompilerParams(
            dimension_semantics=("arbitrary", "arbitrary")),
    )(rowsplits, pred_beta, pred_ccoords, pred_energy, pred_pos, pred_time,
      pred_id, t_idx, t_energy, tt)
    return out[0, 0]


# speedup vs baseline: 1.1659x; 1.1659x over previous
import functools

import jax
import jax.numpy as jnp
from jax.experimental import pallas as pl
from jax.experimental.pallas import tpu as pltpu

K_PER = 256
Q_MIN = 0.5
S_B = 1.0
HUBER_SCALE = 2.0
E_DEN_OFF = 1.0
PAYLOAD_REL_THR = 0.1


def _atanh(x):
    return 0.5 * jnp.log((1.0 + x) / (1.0 - x))


def _oc_kernel(rs_ref, beta_ref, cc_ref, pe_ref, pp_ref, pt_ref, pid_ref,
               tidx_ref, te_ref, tt_ref,
               out_ref,
               mb_ref, mx0_ref, mx1_ref, qa_ref, thr_ref,
               acc_ref, pws_ref, pww_ref, ssum_ref,
               *, B, K, NB, N):
    p = pl.program_id(0)
    b = pl.program_id(1)

    off = jnp.where(b * B >= rs_ref[1], K_PER, 0)
    kio = jax.lax.broadcasted_iota(jnp.int32, (1, K_PER), 1)
    tid = tidx_ref[:, 0:1]
    beta = jnp.clip(beta_ref[:, 0:1], 1e-6, 1.0 - 1e-4)
    M = tid == kio
    x0 = cc_ref[:, 0:1]
    x1 = cc_ref[:, 1:2]
    ksl = pl.ds(off, K_PER)

    @pl.when(p == 0)
    def _phase0():
        @pl.when(b == 0)
        def _init():
            mb_ref[...] = jnp.full((1, K), -1.0, jnp.float32)
            mx0_ref[...] = jnp.zeros((1, K), jnp.float32)
            mx1_ref[...] = jnp.zeros((1, K), jnp.float32)
            acc_ref[...] = jnp.zeros((1, K), jnp.float32)
            pws_ref[...] = jnp.zeros((1, K), jnp.float32)
            pww_ref[...] = jnp.zeros((1, K), jnp.float32)
            ssum_ref[0] = 0.0
            ssum_ref[1] = 0.0

        mbk = jnp.where(M, beta, -1.0)
        bmax = jnp.max(mbk, axis=0, keepdims=True)
        ii = jax.lax.broadcasted_iota(jnp.int32, (B, K_PER), 0)
        win = M & (mbk == bmax)
        iwin = jnp.min(jnp.where(win, ii, N), axis=0, keepdims=True)
        uniq = win & (ii == iwin)
        bx0 = jnp.sum(jnp.where(uniq, x0, 0.0), axis=0, keepdims=True)
        bx1 = jnp.sum(jnp.where(uniq, x1, 0.0), axis=0, keepdims=True)
        upd = bmax > mb_ref[0:1, ksl]
        mb_ref[0:1, ksl] = jnp.where(upd, bmax, mb_ref[0:1, ksl])
        mx0_ref[0:1, ksl] = jnp.where(upd, bx0, mx0_ref[0:1, ksl])
        mx1_ref[0:1, ksl] = jnp.where(upd, bx1, mx1_ref[0:1, ksl])

        nmask = tid < 0
        ssum_ref[0] += jnp.sum(jnp.where(nmask, beta, 0.0))
        ssum_ref[1] += jnp.sum(nmask.astype(jnp.float32))

    @pl.when(p == 1)
    def _phase1():
        @pl.when(b == 0)
        def _prep():
            mb = mb_ref[...]
            exists = mb > 0.0
            ba = jnp.where(exists, mb, 0.0)
            at = _atanh(ba)
            qa_ref[...] = jnp.where(exists, at * at + Q_MIN, 0.0)
            thr_ref[...] = PAYLOAD_REL_THR * ba

        dx0 = x0 - mx0_ref[0:1, ksl]
        dx1 = x1 - mx1_ref[0:1, ksl]
        d2 = dx0 * dx0 + dx1 * dx1
        d = jnp.sqrt(d2 + 1e-9)
        rw = jnp.maximum(1.0 - d, 0.0)
        sel = jnp.where(M, d2, rw)
        at = _atanh(beta)
        q = at * at + Q_MIN
        acc_ref[0:1, ksl] += jnp.sum((q * qa_ref[0:1, ksl]) * sel,
                                     axis=0, keepdims=True)

        te = te_ref[:, 0:1]
        ew = jnp.where(te > 10.0, 1.0, (te / 10.0 + 0.1) / 1.1)
        ste = jnp.sqrt(te + 0.001)
        l = jnp.abs(te - pe_ref[:, 0:1]) / (ste + E_DEN_OFF)
        delta = ste * HUBER_SCALE
        el = jnp.where(l <= delta, 0.5 * l * l, delta * (l - 0.5 * delta))
        dp0 = tt_ref[:, 1:2] - pp_ref[:, 0:1]
        dp1 = tt_ref[:, 2:3] - pp_ref[:, 1:2]
        posl = (dp0 * dp0 + dp1 * dp1) / 100.0
        dtim = tt_ref[:, 0:1] * 1e9 - pt_ref[:, 0:1]
        timl = dtim * dtim
        cls = (1e-8 / 6.0) * jnp.sum(pid_ref[...] * pid_ref[...], axis=1,
                                     keepdims=True)
        wsum = (el + posl + timl + cls) * ew

        pwv = jnp.where(M & (beta > thr_ref[0:1, ksl]), beta, 0.0)
        pws_ref[0:1, ksl] += jnp.sum(pwv, axis=0, keepdims=True)
        pww_ref[0:1, ksl] += jnp.sum(pwv * wsum, axis=0, keepdims=True)

        @pl.when(b == NB - 1)
        def _fin():
            mb = mb_ref[...]
            exists = (mb > 0.0).astype(jnp.float32)
            n_obj = jnp.maximum(jnp.sum(exists), 1.0)
            l_minb = jnp.sum(exists * (1.0 - mb)) / n_obj
            l_pay = jnp.sum(exists * pww_ref[...]
                            / (pws_ref[...] + 1e-9)) / n_obj
            pair = jnp.sum(acc_ref[...]) / N
            l_noise = S_B * ssum_ref[0] / jnp.maximum(ssum_ref[1], 1.0)
            out_ref[...] = (pair + l_minb + l_noise + l_pay).reshape(1, 1)


def kernel(pred_beta, pred_ccoords, pred_energy, pred_pos, pred_time,
           pred_id, t_idx, t_energy, t_pos, t_time, t_pid, rowsplits):
    n = pred_beta.shape[0]
    n_events = rowsplits.shape[0] - 1
    k_tot = n_events * K_PER
    B = 1024
    NB = n // B
    tt = jnp.concatenate([t_time, t_pos], axis=1)

    hspec = lambda c: pl.BlockSpec((B, c), lambda p, b: (b, 0))
    out = pl.pallas_call(
        functools.partial(_oc_kernel, B=B, K=k_tot, NB=NB, N=n),
        grid=(2, NB),
        in_specs=[
            pl.BlockSpec(memory_space=pltpu.SMEM),
            hspec(1), hspec(2), hspec(1), hspec(2), hspec(1), hspec(6),
            hspec(1), hspec(1), hspec(3),
        ],
        out_specs=pl.BlockSpec((1, 1), lambda p, b: (0, 0)),
        out_shape=jax.ShapeDtypeStruct((1, 1), jnp.float32),
        scratch_shapes=[
            pltpu.VMEM((1, k_tot), jnp.float32),
            pltpu.VMEM((1, k_tot), jnp.float32),
            pltpu.VMEM((1, k_tot), jnp.float32),
            pltpu.VMEM((1, k_tot), jnp.float32),
            pltpu.VMEM((1, k_tot), jnp.float32),
            pltpu.VMEM((1, k_tot), jnp.float32),
            pltpu.VMEM((1, k_tot), jnp.float32),
            pltpu.VMEM((1, k_tot), jnp.float32),
            pltpu.SMEM((2,), jnp.float32),
        ],
        compiler_params=pltpu.CompilerParams(
            dimension_semantics=("arbitrary", "arbitrary")),
    )(rowsplits, pred_beta, pred_ccoords, pred_energy, pred_pos, pred_time,
      pred_id, t_idx, t_energy, tt)
    return out[0, 0]
